# Initial kernel scaffold; baseline (speedup 1.0000x reference)
import functools

import jax
import jax.numpy as jnp
from jax import lax
from jax.experimental import pallas as pl
from jax.experimental.pallas import tpu as pltpu
from jax.experimental.pallas import tpu_sc as plsc

MF_DIM = 32
WORD_DIM = 64
DOC_LEN = 200
HALF = DOC_LEN // 2
B = 4096
NC, NS = 2, 16
NW = NC * NS
BPW = B // NW
SCALE = 1.0 / DOC_LEN


def _accum_doc(rows_ref):
    zero = jnp.zeros((16,), jnp.float32)

    def body(j, acc):
        a0, a1, a2, a3 = acc
        return (
            a0 + rows_ref[j, pl.ds(0, 16)],
            a1 + rows_ref[j, pl.ds(16, 16)],
            a2 + rows_ref[j, pl.ds(32, 16)],
            a3 + rows_ref[j, pl.ds(48, 16)],
        )

    return lax.fori_loop(0, DOC_LEN, body, (zero, zero, zero, zero))


def _enc_body(user_hbm, item_hbm, user_doc_hbm, item_doc_hbm,
              gu_w, gi_w, tu_w, ti_w, doc_w,
              o_gu, o_gi, o_tu, o_ti, o_ud, o_id,
              idx_v, rows_v, doc_idx_v, doc_rows_v, out_doc_v, sem, dsem):
    wid = lax.axis_index("s") * NC + lax.axis_index("c")
    base = wid * BPW

    pltpu.sync_copy(user_hbm.at[pl.ds(base, BPW)], idx_v)
    pltpu.async_copy(gu_w.at[idx_v], rows_v, sem).wait()
    pltpu.sync_copy(rows_v, o_gu.at[pl.ds(base, BPW)])
    pltpu.async_copy(tu_w.at[idx_v], rows_v, sem).wait()
    pltpu.sync_copy(rows_v, o_tu.at[pl.ds(base, BPW)])

    pltpu.sync_copy(item_hbm.at[pl.ds(base, BPW)], idx_v)
    pltpu.async_copy(gi_w.at[idx_v], rows_v, sem).wait()
    pltpu.sync_copy(rows_v, o_gi.at[pl.ds(base, BPW)])
    pltpu.async_copy(ti_w.at[idx_v], rows_v, sem).wait()
    pltpu.sync_copy(rows_v, o_ti.at[pl.ds(base, BPW)])

    for doc_hbm, o_doc in ((user_doc_hbm, o_ud), (item_doc_hbm, o_id)):
        pltpu.sync_copy(doc_hbm.at[pl.ds(base * 2, 2 * BPW)], doc_idx_v)

        def elem(e, _, doc_idx_v=doc_idx_v, doc_rows_v=doc_rows_v,
                 out_doc_v=out_doc_v, dsem=dsem):
            d0 = pltpu.async_copy(
                doc_w.at[doc_idx_v.at[2 * e]],
                doc_rows_v.at[pl.ds(0, HALF)], dsem)
            d1 = pltpu.async_copy(
                doc_w.at[doc_idx_v.at[2 * e + 1]],
                doc_rows_v.at[pl.ds(HALF, HALF)], dsem)
            d0.wait()
            d1.wait()
            a0, a1, a2, a3 = _accum_doc(doc_rows_v)
            out_doc_v[e, pl.ds(0, 16)] = a0 * SCALE
            out_doc_v[e, pl.ds(16, 16)] = a1 * SCALE
            out_doc_v[e, pl.ds(32, 16)] = a2 * SCALE
            out_doc_v[e, pl.ds(48, 16)] = a3 * SCALE
            return 0

        lax.fori_loop(0, BPW, elem, 0)
        pltpu.sync_copy(out_doc_v, o_doc.at[pl.ds(base, BPW)])


@jax.jit
def _encoder_call(user, item, user_doc2, item_doc2,
                  gamma_user_w, gamma_item_w, theta_user_w, theta_item_w,
                  doc_w):
    mesh = plsc.VectorSubcoreMesh(core_axis_name="c", subcore_axis_name="s")
    out_type = (
        jax.ShapeDtypeStruct((B, MF_DIM), jnp.float32),
        jax.ShapeDtypeStruct((B, MF_DIM), jnp.float32),
        jax.ShapeDtypeStruct((B, MF_DIM), jnp.float32),
        jax.ShapeDtypeStruct((B, MF_DIM), jnp.float32),
        jax.ShapeDtypeStruct((B, WORD_DIM), jnp.float32),
        jax.ShapeDtypeStruct((B, WORD_DIM), jnp.float32),
    )
    scratch = [
        pltpu.VMEM((BPW,), jnp.int32),
        pltpu.VMEM((BPW, MF_DIM), jnp.float32),
        pltpu.VMEM((2 * BPW, HALF), jnp.int32),
        pltpu.VMEM((DOC_LEN, WORD_DIM), jnp.float32),
        pltpu.VMEM((BPW, WORD_DIM), jnp.float32),
        pltpu.SemaphoreType.DMA,
        pltpu.SemaphoreType.DMA,
    ]
    run = pl.kernel(_enc_body, out_type=out_type, mesh=mesh,
                    scratch_types=scratch)
    return run(user, item, user_doc2, item_doc2,
               gamma_user_w, gamma_item_w, theta_user_w, theta_item_w, doc_w)


def kernel(user, item, user_doc, item_doc, gamma_user_w, gamma_item_w,
           theta_user_w, theta_item_w, doc_w):
    user_doc2 = user_doc.reshape(2 * B, HALF)
    item_doc2 = item_doc.reshape(2 * B, HALF)
    return _encoder_call(user, item, user_doc2, item_doc2,
                         gamma_user_w, gamma_item_w, theta_user_w,
                         theta_item_w, doc_w)


# speedup vs baseline: 3.9153x; 3.9153x over previous
import functools

import jax
import jax.numpy as jnp
from jax import lax
from jax.experimental import pallas as pl
from jax.experimental.pallas import tpu as pltpu
from jax.experimental.pallas import tpu_sc as plsc

MF_DIM = 32
WORD_DIM = 64
DOC_LEN = 200
HALF = DOC_LEN // 2
B = 4096
NC, NS = 2, 16
NW = NC * NS
BPW = B // NW
SCALE = 1.0 / DOC_LEN


def _accum_doc(rows_ref):
    zero = jnp.zeros((16,), jnp.float32)

    def body(j, acc):
        a0, a1, a2, a3 = acc
        return (
            a0 + rows_ref[j, pl.ds(0, 16)],
            a1 + rows_ref[j, pl.ds(16, 16)],
            a2 + rows_ref[j, pl.ds(32, 16)],
            a3 + rows_ref[j, pl.ds(48, 16)],
        )

    return lax.fori_loop(0, DOC_LEN, body, (zero, zero, zero, zero))


def _enc_body(user_hbm, item_hbm, user_doc_hbm, item_doc_hbm,
              gu_w, gi_w, tu_w, ti_w, doc_w,
              o_gu, o_gi, o_tu, o_ti, o_ud, o_id,
              idx_v, rows_v, doc_idx_v, doc_rows_v, out_doc_v, sem, dsem):
    wid = lax.axis_index("s") * NC + lax.axis_index("c")
    base = wid * BPW

    pltpu.sync_copy(user_hbm.at[pl.ds(base, BPW)], idx_v)
    pltpu.async_copy(gu_w.at[idx_v], rows_v, sem).wait()
    pltpu.sync_copy(rows_v, o_gu.at[pl.ds(base, BPW)])
    pltpu.async_copy(tu_w.at[idx_v], rows_v, sem).wait()
    pltpu.sync_copy(rows_v, o_tu.at[pl.ds(base, BPW)])

    pltpu.sync_copy(item_hbm.at[pl.ds(base, BPW)], idx_v)
    pltpu.async_copy(gi_w.at[idx_v], rows_v, sem).wait()
    pltpu.sync_copy(rows_v, o_gi.at[pl.ds(base, BPW)])
    pltpu.async_copy(ti_w.at[idx_v], rows_v, sem).wait()
    pltpu.sync_copy(rows_v, o_ti.at[pl.ds(base, BPW)])

    for doc_hbm, o_doc in ((user_doc_hbm, o_ud), (item_doc_hbm, o_id)):
        pltpu.sync_copy(doc_hbm.at[pl.ds(base * 2, 2 * BPW)], doc_idx_v)

        def elem(e, _, doc_idx_v=doc_idx_v, doc_rows_v=doc_rows_v,
                 out_doc_v=out_doc_v, dsem=dsem):
            d0 = pltpu.async_copy(
                doc_w.at[doc_idx_v.at[2 * e]],
                doc_rows_v.at[pl.ds(0, HALF)], dsem)
            d1 = pltpu.async_copy(
                doc_w.at[doc_idx_v.at[2 * e + 1]],
                doc_rows_v.at[pl.ds(HALF, HALF)], dsem)
            d0.wait()
            d1.wait()
            a0, a1, a2, a3 = _accum_doc(doc_rows_v)
            out_doc_v[e, pl.ds(0, 16)] = a0 * SCALE
            out_doc_v[e, pl.ds(16, 16)] = a1 * SCALE
            out_doc_v[e, pl.ds(32, 16)] = a2 * SCALE
            out_doc_v[e, pl.ds(48, 16)] = a3 * SCALE
            return 0

        lax.fori_loop(0, BPW, elem, 0)
        pltpu.sync_copy(out_doc_v, o_doc.at[pl.ds(base, BPW)])


@jax.jit
def _encoder_call(user, item, user_doc2, item_doc2,
                  gamma_user_w, gamma_item_w, theta_user_w, theta_item_w,
                  doc_w):
    mesh = plsc.VectorSubcoreMesh(core_axis_name="c", subcore_axis_name="s",
                                  num_cores=NC, num_subcores=NS)
    out_type = (
        jax.ShapeDtypeStruct((B, MF_DIM), jnp.float32),
        jax.ShapeDtypeStruct((B, MF_DIM), jnp.float32),
        jax.ShapeDtypeStruct((B, MF_DIM), jnp.float32),
        jax.ShapeDtypeStruct((B, MF_DIM), jnp.float32),
        jax.ShapeDtypeStruct((B, WORD_DIM), jnp.float32),
        jax.ShapeDtypeStruct((B, WORD_DIM), jnp.float32),
    )
    scratch = [
        pltpu.VMEM((BPW,), jnp.int32),
        pltpu.VMEM((BPW, MF_DIM), jnp.float32),
        pltpu.VMEM((2 * BPW, HALF), jnp.int32),
        pltpu.VMEM((DOC_LEN, WORD_DIM), jnp.float32),
        pltpu.VMEM((BPW, WORD_DIM), jnp.float32),
        pltpu.SemaphoreType.DMA,
        pltpu.SemaphoreType.DMA,
    ]
    run = pl.kernel(_enc_body, out_type=out_type, mesh=mesh,
                    scratch_types=scratch,
                    compiler_params=pltpu.CompilerParams(
                        use_tc_tiling_on_sc=False))
    return run(user, item, user_doc2, item_doc2,
               gamma_user_w, gamma_item_w, theta_user_w, theta_item_w, doc_w)


def kernel(user, item, user_doc, item_doc, gamma_user_w, gamma_item_w,
           theta_user_w, theta_item_w, doc_w):
    user_doc2 = user_doc.reshape(2 * B, HALF)
    item_doc2 = item_doc.reshape(2 * B, HALF)
    return _encoder_call(user, item, user_doc2, item_doc2,
                         gamma_user_w, gamma_item_w, theta_user_w,
                         theta_item_w, doc_w)


# speedup vs baseline: 4.7189x; 1.2052x over previous
import functools

import jax
import jax.numpy as jnp
from jax import lax
from jax.experimental import pallas as pl
from jax.experimental.pallas import tpu as pltpu
from jax.experimental.pallas import tpu_sc as plsc

MF_DIM = 32
WORD_DIM = 64
DOC_LEN = 200
HALF = DOC_LEN // 2
B = 4096
NC, NS = 2, 16
NW = NC * NS
BPW = B // NW
SCALE = 1.0 / DOC_LEN


UNROLL = 8


def _accum_doc(rows_ref):
    zero = jnp.zeros((16,), jnp.float32)

    def body(j, acc):
        row0 = j * UNROLL
        acc = list(acc)
        for r in range(UNROLL):
            for c in range(4):
                k = (r % 2) * 4 + c
                acc[k] = acc[k] + rows_ref[row0 + r, pl.ds(c * 16, 16)]
        return tuple(acc)

    acc = lax.fori_loop(0, DOC_LEN // UNROLL, body, (zero,) * 8)
    return tuple(acc[c] + acc[4 + c] for c in range(4))


def _enc_body(user_hbm, item_hbm, user_doc_hbm, item_doc_hbm,
              gu_w, gi_w, tu_w, ti_w, doc_w,
              o_gu, o_gi, o_tu, o_ti, o_ud, o_id,
              idx_v, rows_v, doc_idx_v, doc_rows_v, out_doc_v,
              sem, dsem_a, dsem_b):
    wid = lax.axis_index("s") * NC + lax.axis_index("c")
    base = wid * BPW

    pltpu.sync_copy(user_hbm.at[pl.ds(base, BPW)], idx_v)
    pltpu.async_copy(gu_w.at[idx_v], rows_v, sem).wait()
    pltpu.sync_copy(rows_v, o_gu.at[pl.ds(base, BPW)])
    pltpu.async_copy(tu_w.at[idx_v], rows_v, sem).wait()
    pltpu.sync_copy(rows_v, o_tu.at[pl.ds(base, BPW)])

    pltpu.sync_copy(item_hbm.at[pl.ds(base, BPW)], idx_v)
    pltpu.async_copy(gi_w.at[idx_v], rows_v, sem).wait()
    pltpu.sync_copy(rows_v, o_gi.at[pl.ds(base, BPW)])
    pltpu.async_copy(ti_w.at[idx_v], rows_v, sem).wait()
    pltpu.sync_copy(rows_v, o_ti.at[pl.ds(base, BPW)])

    buf_a = doc_rows_v.at[0]
    buf_b = doc_rows_v.at[1]

    def fire(e, buf, sem):
        pltpu.async_copy(doc_w.at[doc_idx_v.at[2 * e]],
                         buf.at[pl.ds(0, HALF)], sem)
        pltpu.async_copy(doc_w.at[doc_idx_v.at[2 * e + 1]],
                         buf.at[pl.ds(HALF, HALF)], sem)

    def drain(buf, sem):
        pltpu.make_async_copy(doc_w.at[pl.ds(0, DOC_LEN)], buf, sem).wait()

    def pool(e, buf):
        a0, a1, a2, a3 = _accum_doc(buf)
        out_doc_v[e, pl.ds(0, 16)] = a0 * SCALE
        out_doc_v[e, pl.ds(16, 16)] = a1 * SCALE
        out_doc_v[e, pl.ds(32, 16)] = a2 * SCALE
        out_doc_v[e, pl.ds(48, 16)] = a3 * SCALE

    for doc_hbm, o_doc in ((user_doc_hbm, o_ud), (item_doc_hbm, o_id)):
        pltpu.sync_copy(doc_hbm.at[pl.ds(base * 2, 2 * BPW)], doc_idx_v)
        fire(0, buf_a, dsem_a)

        def step(i, _):
            fire(2 * i + 1, buf_b, dsem_b)
            drain(buf_a, dsem_a)
            pool(2 * i, buf_a)

            @pl.when(i < BPW // 2 - 1)
            def _():
                fire(2 * i + 2, buf_a, dsem_a)

            drain(buf_b, dsem_b)
            pool(2 * i + 1, buf_b)
            return 0

        lax.fori_loop(0, BPW // 2, step, 0)
        pltpu.sync_copy(out_doc_v, o_doc.at[pl.ds(base, BPW)])


@jax.jit
def _encoder_call(user, item, user_doc2, item_doc2,
                  gamma_user_w, gamma_item_w, theta_user_w, theta_item_w,
                  doc_w):
    mesh = plsc.VectorSubcoreMesh(core_axis_name="c", subcore_axis_name="s",
                                  num_cores=NC, num_subcores=NS)
    out_type = (
        jax.ShapeDtypeStruct((B, MF_DIM), jnp.float32),
        jax.ShapeDtypeStruct((B, MF_DIM), jnp.float32),
        jax.ShapeDtypeStruct((B, MF_DIM), jnp.float32),
        jax.ShapeDtypeStruct((B, MF_DIM), jnp.float32),
        jax.ShapeDtypeStruct((B, WORD_DIM), jnp.float32),
        jax.ShapeDtypeStruct((B, WORD_DIM), jnp.float32),
    )
    scratch = [
        pltpu.VMEM((BPW,), jnp.int32),
        pltpu.VMEM((BPW, MF_DIM), jnp.float32),
        pltpu.VMEM((2 * BPW, HALF), jnp.int32),
        pltpu.VMEM((2, DOC_LEN, WORD_DIM), jnp.float32),
        pltpu.VMEM((BPW, WORD_DIM), jnp.float32),
        pltpu.SemaphoreType.DMA,
        pltpu.SemaphoreType.DMA,
        pltpu.SemaphoreType.DMA,
    ]
    run = pl.kernel(_enc_body, out_type=out_type, mesh=mesh,
                    scratch_types=scratch,
                    compiler_params=pltpu.CompilerParams(
                        use_tc_tiling_on_sc=False))
    return run(user, item, user_doc2, item_doc2,
               gamma_user_w, gamma_item_w, theta_user_w, theta_item_w, doc_w)


def kernel(user, item, user_doc, item_doc, gamma_user_w, gamma_item_w,
           theta_user_w, theta_item_w, doc_w):
    user_doc2 = user_doc.reshape(2 * B, HALF)
    item_doc2 = item_doc.reshape(2 * B, HALF)
    return _encoder_call(user, item, user_doc2, item_doc2,
                         gamma_user_w, gamma_item_w, theta_user_w,
                         theta_item_w, doc_w)


# speedup vs baseline: 10.9764x; 2.3261x over previous
import functools

import jax
import jax.numpy as jnp
from jax import lax
from jax.experimental import pallas as pl
from jax.experimental.pallas import tpu as pltpu
from jax.experimental.pallas import tpu_sc as plsc

MF_DIM = 32
WORD_DIM = 64
DOC_LEN = 200
HALF = DOC_LEN // 2
B = 4096
NC, NS = 2, 16
NW = NC * NS
BPW = B // NW
SCALE = 1.0 / DOC_LEN
UNROLL = 8


def _worker_base():
    wid = lax.axis_index("s") * NC + lax.axis_index("c")
    return wid * BPW


def _accum_doc(rows_ref):
    zero = jnp.zeros((16,), jnp.float32)

    def body(j, acc):
        row0 = j * UNROLL
        acc = list(acc)
        for r in range(UNROLL):
            for c in range(4):
                k = (r % 2) * 4 + c
                acc[k] = acc[k] + rows_ref[row0 + r, pl.ds(c * 16, 16)]
        return tuple(acc)

    acc = lax.fori_loop(0, DOC_LEN // UNROLL, body, (zero,) * 8)
    return tuple(acc[c] + acc[4 + c] for c in range(4))


def _doc_body(user_doc_hbm, item_doc_hbm, doc_w,
              o_ud, o_id,
              doc_idx_v, doc_rows_v, out_doc_v, dsem_a, dsem_b):
    base = _worker_base()
    buf_a = doc_rows_v.at[0]
    buf_b = doc_rows_v.at[1]

    def fire(e, buf, sem):
        pltpu.async_copy(doc_w.at[doc_idx_v.at[2 * e]],
                         buf.at[pl.ds(0, HALF)], sem)
        pltpu.async_copy(doc_w.at[doc_idx_v.at[2 * e + 1]],
                         buf.at[pl.ds(HALF, HALF)], sem)

    def drain(buf, sem):
        pltpu.make_async_copy(doc_w.at[pl.ds(0, DOC_LEN)], buf, sem).wait()

    def pool(e, buf):
        a0, a1, a2, a3 = _accum_doc(buf)
        out_doc_v[e, pl.ds(0, 16)] = a0 * SCALE
        out_doc_v[e, pl.ds(16, 16)] = a1 * SCALE
        out_doc_v[e, pl.ds(32, 16)] = a2 * SCALE
        out_doc_v[e, pl.ds(48, 16)] = a3 * SCALE

    for doc_hbm, o_doc in ((user_doc_hbm, o_ud), (item_doc_hbm, o_id)):
        pltpu.sync_copy(doc_hbm.at[pl.ds(base * 2, 2 * BPW)], doc_idx_v)
        fire(0, buf_a, dsem_a)

        def step(i, _):
            fire(2 * i + 1, buf_b, dsem_b)
            drain(buf_a, dsem_a)
            pool(2 * i, buf_a)

            @pl.when(i < BPW // 2 - 1)
            def _():
                fire(2 * i + 2, buf_a, dsem_a)

            drain(buf_b, dsem_b)
            pool(2 * i + 1, buf_b)
            return 0

        lax.fori_loop(0, BPW // 2, step, 0)
        pltpu.sync_copy(out_doc_v, o_doc.at[pl.ds(base, BPW)])


def _mf_body(user_hbm, item_hbm, guT, giT, tuT, tiT,
             o_gu, o_gi, o_tu, o_ti,
             idx_v, blk_v, rows_v, bsem_a, bsem_b):
    base = _worker_base()
    lanes = lax.iota(jnp.int32, 16)

    def sidx(e):
        return idx_v[pl.ds(e, 16)][0]

    def fire(tblT, e, slot, sem):
        i = sidx(e)
        start = pl.multiple_of((i // 128) * 128, 128)
        pltpu.async_copy(tblT.at[:, pl.ds(start, 128)],
                         blk_v.at[slot], sem)

    def drain(tblT, slot, sem):
        pltpu.make_async_copy(tblT.at[:, pl.ds(0, 128)], blk_v.at[slot],
                              sem).wait()

    def extract(e, slot):
        lane = jnp.full((16,), sidx(e) % 128, jnp.int32)
        rows_v[e, pl.ds(0, 16)] = plsc.load_gather(
            blk_v.at[slot], [lanes, lane])
        rows_v[e, pl.ds(16, 16)] = plsc.load_gather(
            blk_v.at[slot], [lanes + 16, lane])

    for idx_hbm, tblT, o in ((user_hbm, guT, o_gu), (item_hbm, giT, o_gi),
                             (user_hbm, tuT, o_tu), (item_hbm, tiT, o_ti)):
        pltpu.sync_copy(idx_hbm.at[pl.ds(base, BPW)],
                        idx_v.at[pl.ds(0, BPW)])
        fire(tblT, 0, 0, bsem_a)

        def step(i, _, tblT=tblT):
            fire(tblT, 2 * i + 1, 1, bsem_b)
            drain(tblT, 0, bsem_a)
            extract(2 * i, 0)

            @pl.when(i < BPW // 2 - 1)
            def _():
                fire(tblT, 2 * i + 2, 0, bsem_a)

            drain(tblT, 1, bsem_b)
            extract(2 * i + 1, 1)
            return 0

        lax.fori_loop(0, BPW // 2, step, 0)
        pltpu.sync_copy(rows_v, o.at[pl.ds(base, BPW)])


@jax.jit
def _encoder_call(user, item, user_doc2, item_doc2,
                  gamma_user_w, gamma_item_w, theta_user_w, theta_item_w,
                  doc_w):
    mesh = plsc.VectorSubcoreMesh(core_axis_name="c", subcore_axis_name="s",
                                  num_cores=NC, num_subcores=NS)

    doc_out = (
        jax.ShapeDtypeStruct((B, WORD_DIM), jnp.float32),
        jax.ShapeDtypeStruct((B, WORD_DIM), jnp.float32),
    )
    doc_scratch = [
        pltpu.VMEM((2 * BPW, HALF), jnp.int32),
        pltpu.VMEM((2, DOC_LEN, WORD_DIM), jnp.float32),
        pltpu.VMEM((BPW, WORD_DIM), jnp.float32),
        pltpu.SemaphoreType.DMA,
        pltpu.SemaphoreType.DMA,
    ]
    doc_run = pl.kernel(_doc_body, out_type=doc_out, mesh=mesh,
                        scratch_types=doc_scratch,
                        compiler_params=pltpu.CompilerParams(
                            use_tc_tiling_on_sc=False))
    ud_embed, id_embed = doc_run(user_doc2, item_doc2, doc_w)

    mf_out = tuple(
        jax.ShapeDtypeStruct((B, MF_DIM), jnp.float32) for _ in range(4))
    mf_scratch = [
        pltpu.VMEM((BPW + 16,), jnp.int32),
        pltpu.VMEM((2, MF_DIM, 128), jnp.float32),
        pltpu.VMEM((BPW, MF_DIM), jnp.float32),
        pltpu.SemaphoreType.DMA,
        pltpu.SemaphoreType.DMA,
    ]
    mf_run = pl.kernel(_mf_body, out_type=mf_out, mesh=mesh,
                       scratch_types=mf_scratch,
                       compiler_params=pltpu.CompilerParams(
                           use_tc_tiling_on_sc=True,
                           disable_bounds_checks=True,
                           needs_layout_passes=False))
    gu, gi, tu, ti = mf_run(user, item, gamma_user_w.T, gamma_item_w.T,
                            theta_user_w.T, theta_item_w.T)
    return gu, gi, tu, ti, ud_embed, id_embed


def kernel(user, item, user_doc, item_doc, gamma_user_w, gamma_item_w,
           theta_user_w, theta_item_w, doc_w):
    user_doc2 = user_doc.reshape(2 * B, HALF)
    item_doc2 = item_doc.reshape(2 * B, HALF)
    return _encoder_call(user, item, user_doc2, item_doc2,
                         gamma_user_w, gamma_item_w, theta_user_w,
                         theta_item_w, doc_w)


# speedup vs baseline: 16.1827x; 1.4743x over previous
import functools

import jax
import jax.numpy as jnp
from jax import lax
from jax.experimental import pallas as pl
from jax.experimental.pallas import tpu as pltpu
from jax.experimental.pallas import tpu_sc as plsc

MF_DIM = 32
WORD_DIM = 64
DOC_LEN = 200
HALF = DOC_LEN // 2
B = 4096
NC, NS = 2, 16
NW = NC * NS
BPW = B // NW
SCALE = 1.0 / DOC_LEN
UNROLL = 8
DOC_NBUF = 4
MF_NBUF = 8


def _worker_base():
    wid = lax.axis_index("s") * NC + lax.axis_index("c")
    return wid * BPW


def _accum_doc(rows_ref):
    zero = jnp.zeros((16,), jnp.float32)

    def body(j, acc):
        row0 = j * UNROLL
        acc = list(acc)
        for r in range(UNROLL):
            for c in range(4):
                k = (r % 2) * 4 + c
                acc[k] = acc[k] + rows_ref[row0 + r, pl.ds(c * 16, 16)]
        return tuple(acc)

    acc = lax.fori_loop(0, DOC_LEN // UNROLL, body, (zero,) * 8)
    return tuple(acc[c] + acc[4 + c] for c in range(4))


def _doc_body(user_hbm, user_doc_hbm, item_doc_hbm, gu_w, tu_w, doc_w,
              o_gu, o_tu, o_ud, o_id,
              uidx_v, mfrows_v, doc_idx_v, doc_rows_v, out_doc_v,
              usem, dsem0, dsem1, dsem2, dsem3):
    base = _worker_base()
    dsems = (dsem0, dsem1, dsem2, dsem3)

    pltpu.sync_copy(user_hbm.at[pl.ds(base, BPW)], uidx_v)
    pltpu.async_copy(gu_w.at[uidx_v], mfrows_v, usem).wait()
    pltpu.sync_copy(mfrows_v, o_gu.at[pl.ds(base, BPW)])
    pltpu.async_copy(tu_w.at[uidx_v], mfrows_v, usem).wait()
    pltpu.sync_copy(mfrows_v, o_tu.at[pl.ds(base, BPW)])

    def fire(e, s):
        buf = doc_rows_v.at[s]
        pltpu.async_copy(doc_w.at[doc_idx_v.at[2 * e]],
                         buf.at[pl.ds(0, HALF)], dsems[s])
        pltpu.async_copy(doc_w.at[doc_idx_v.at[2 * e + 1]],
                         buf.at[pl.ds(HALF, HALF)], dsems[s])

    def drain(s):
        pltpu.make_async_copy(doc_w.at[pl.ds(0, DOC_LEN)], doc_rows_v.at[s],
                              dsems[s]).wait()

    def pool(e, s):
        a0, a1, a2, a3 = _accum_doc(doc_rows_v.at[s])
        out_doc_v[e, pl.ds(0, 16)] = a0 * SCALE
        out_doc_v[e, pl.ds(16, 16)] = a1 * SCALE
        out_doc_v[e, pl.ds(32, 16)] = a2 * SCALE
        out_doc_v[e, pl.ds(48, 16)] = a3 * SCALE

    for doc_hbm, o_doc in ((user_doc_hbm, o_ud), (item_doc_hbm, o_id)):
        pltpu.sync_copy(doc_hbm.at[pl.ds(base * 2, 2 * BPW)], doc_idx_v)
        for s in range(DOC_NBUF):
            fire(s, s)

        def group(g, _):
            for s in range(DOC_NBUF):
                e = g * DOC_NBUF + s
                drain(s)
                pool(e, s)

                @pl.when(e + DOC_NBUF < BPW)
                def _(e=e, s=s):
                    fire(e + DOC_NBUF, s)
            return 0

        lax.fori_loop(0, BPW // DOC_NBUF, group, 0)
        pltpu.sync_copy(out_doc_v, o_doc.at[pl.ds(base, BPW)])


def _item_body(item_hbm, giT, tiT,
               o_gi, o_ti,
               idx_v, blk_v, rows_v, *bsems):
    base = _worker_base()
    lanes = lax.iota(jnp.int32, 16)

    def sidx(e):
        return idx_v[pl.ds(e, 16)][0]

    def fire(tblT, e, s):
        i = sidx(e)
        start = pl.multiple_of((i // 128) * 128, 128)
        pltpu.async_copy(tblT.at[:, pl.ds(start, 128)],
                         blk_v.at[s], bsems[s])

    def drain(tblT, s):
        pltpu.make_async_copy(tblT.at[:, pl.ds(0, 128)], blk_v.at[s],
                              bsems[s]).wait()

    def extract(e, s):
        lane = jnp.full((16,), sidx(e) % 128, jnp.int32)
        rows_v[e, pl.ds(0, 16)] = plsc.load_gather(
            blk_v.at[s], [lanes, lane])
        rows_v[e, pl.ds(16, 16)] = plsc.load_gather(
            blk_v.at[s], [lanes + 16, lane])

    pltpu.sync_copy(item_hbm.at[pl.ds(base, BPW)], idx_v.at[pl.ds(0, BPW)])
    for tblT, o in ((giT, o_gi), (tiT, o_ti)):
        for s in range(MF_NBUF):
            fire(tblT, s, s)

        def group(g, _, tblT=tblT):
            for s in range(MF_NBUF):
                e = g * MF_NBUF + s
                drain(tblT, s)
                extract(e, s)

                @pl.when(e + MF_NBUF < BPW)
                def _(e=e, s=s, tblT=tblT):
                    fire(tblT, e + MF_NBUF, s)
            return 0

        lax.fori_loop(0, BPW // MF_NBUF, group, 0)
        pltpu.sync_copy(rows_v, o.at[pl.ds(base, BPW)])


@jax.jit
def _encoder_call(user, item, user_doc2, item_doc2,
                  gamma_user_w, gamma_item_w, theta_user_w, theta_item_w,
                  doc_w):
    mesh = plsc.VectorSubcoreMesh(core_axis_name="c", subcore_axis_name="s",
                                  num_cores=NC, num_subcores=NS)

    doc_out = (
        jax.ShapeDtypeStruct((B, MF_DIM), jnp.float32),
        jax.ShapeDtypeStruct((B, MF_DIM), jnp.float32),
        jax.ShapeDtypeStruct((B, WORD_DIM), jnp.float32),
        jax.ShapeDtypeStruct((B, WORD_DIM), jnp.float32),
    )
    doc_scratch = [
        pltpu.VMEM((BPW,), jnp.int32),
        pltpu.VMEM((BPW, MF_DIM), jnp.float32),
        pltpu.VMEM((2 * BPW, HALF), jnp.int32),
        pltpu.VMEM((DOC_NBUF, DOC_LEN, WORD_DIM), jnp.float32),
        pltpu.VMEM((BPW, WORD_DIM), jnp.float32),
    ] + [pltpu.SemaphoreType.DMA] * (1 + DOC_NBUF)
    doc_run = pl.kernel(_doc_body, out_type=doc_out, mesh=mesh,
                        scratch_types=doc_scratch,
                        compiler_params=pltpu.CompilerParams(
                            use_tc_tiling_on_sc=False))
    gu, tu, ud_embed, id_embed = doc_run(
        user, user_doc2, item_doc2, gamma_user_w, theta_user_w, doc_w)

    item_out = (
        jax.ShapeDtypeStruct((B, MF_DIM), jnp.float32),
        jax.ShapeDtypeStruct((B, MF_DIM), jnp.float32),
    )
    item_scratch = [
        pltpu.VMEM((BPW + 16,), jnp.int32),
        pltpu.VMEM((MF_NBUF, MF_DIM, 128), jnp.float32),
        pltpu.VMEM((BPW, MF_DIM), jnp.float32),
    ] + [pltpu.SemaphoreType.DMA] * MF_NBUF
    item_run = pl.kernel(_item_body, out_type=item_out, mesh=mesh,
                         scratch_types=item_scratch,
                         compiler_params=pltpu.CompilerParams(
                             use_tc_tiling_on_sc=True,
                             disable_bounds_checks=True,
                             needs_layout_passes=False))
    gi, ti = item_run(item, gamma_item_w.T, theta_item_w.T)
    return gu, gi, tu, ti, ud_embed, id_embed


def kernel(user, item, user_doc, item_doc, gamma_user_w, gamma_item_w,
           theta_user_w, theta_item_w, doc_w):
    user_doc2 = user_doc.reshape(2 * B, HALF)
    item_doc2 = item_doc.reshape(2 * B, HALF)
    return _encoder_call(user, item, user_doc2, item_doc2,
                         gamma_user_w, gamma_item_w, theta_user_w,
                         theta_item_w, doc_w)


# speedup vs baseline: 16.2501x; 1.0042x over previous
import functools

import jax
import jax.numpy as jnp
from jax import lax
from jax.experimental import pallas as pl
from jax.experimental.pallas import tpu as pltpu
from jax.experimental.pallas import tpu_sc as plsc

MF_DIM = 32
WORD_DIM = 64
DOC_LEN = 200
HALF = DOC_LEN // 2
B = 4096
NC, NS = 2, 16
NW = NC * NS
BPW = B // NW
SCALE = 1.0 / DOC_LEN
UNROLL = 10
DOC_NBUF = 4
MF_NBUF = 8


def _worker_base():
    wid = lax.axis_index("s") * NC + lax.axis_index("c")
    return wid * BPW


def _accum_doc(rows_ref):
    zero = jnp.zeros((16,), jnp.float32)

    def body(j, acc):
        row0 = j * UNROLL
        acc = list(acc)
        for r in range(UNROLL):
            for c in range(4):
                k = (r % 2) * 4 + c
                acc[k] = acc[k] + rows_ref[row0 + r, pl.ds(c * 16, 16)]
        return tuple(acc)

    acc = lax.fori_loop(0, DOC_LEN // UNROLL, body, (zero,) * 8)
    return tuple(acc[c] + acc[4 + c] for c in range(4))


def _doc_body(user_hbm, user_doc_hbm, item_doc_hbm, gu_w, tu_w, doc_w,
              o_gu, o_tu, o_ud, o_id,
              uidx_v, gurows_v, turows_v, doc_idx_v, doc_rows_v, out_doc_v,
              usem, dsem0, dsem1, dsem2, dsem3):
    base = _worker_base()
    dsems = (dsem0, dsem1, dsem2, dsem3)

    pltpu.sync_copy(user_hbm.at[pl.ds(base, BPW)], uidx_v)
    gu_copy = pltpu.async_copy(gu_w.at[uidx_v], gurows_v, usem)
    tu_copy = pltpu.async_copy(tu_w.at[uidx_v], turows_v, usem)

    def fire(e, s):
        buf = doc_rows_v.at[s]
        pltpu.async_copy(doc_w.at[doc_idx_v.at[2 * e]],
                         buf.at[pl.ds(0, HALF)], dsems[s])
        pltpu.async_copy(doc_w.at[doc_idx_v.at[2 * e + 1]],
                         buf.at[pl.ds(HALF, HALF)], dsems[s])

    def drain(s):
        pltpu.make_async_copy(doc_w.at[pl.ds(0, DOC_LEN)], doc_rows_v.at[s],
                              dsems[s]).wait()

    def pool(e, s):
        a0, a1, a2, a3 = _accum_doc(doc_rows_v.at[s])
        out_doc_v[e, pl.ds(0, 16)] = a0 * SCALE
        out_doc_v[e, pl.ds(16, 16)] = a1 * SCALE
        out_doc_v[e, pl.ds(32, 16)] = a2 * SCALE
        out_doc_v[e, pl.ds(48, 16)] = a3 * SCALE

    for doc_hbm, o_doc in ((user_doc_hbm, o_ud), (item_doc_hbm, o_id)):
        pltpu.sync_copy(doc_hbm.at[pl.ds(base * 2, 2 * BPW)], doc_idx_v)
        for s in range(DOC_NBUF):
            fire(s, s)

        def group(g, _):
            for s in range(DOC_NBUF):
                e = g * DOC_NBUF + s
                drain(s)
                pool(e, s)

                @pl.when(e + DOC_NBUF < BPW)
                def _(e=e, s=s):
                    fire(e + DOC_NBUF, s)
            return 0

        lax.fori_loop(0, BPW // DOC_NBUF, group, 0)
        pltpu.sync_copy(out_doc_v, o_doc.at[pl.ds(base, BPW)])

    gu_copy.wait()
    tu_copy.wait()
    pltpu.sync_copy(gurows_v, o_gu.at[pl.ds(base, BPW)])
    pltpu.sync_copy(turows_v, o_tu.at[pl.ds(base, BPW)])


def _item_body(item_hbm, giT, tiT,
               o_gi, o_ti,
               idx_v, blk_v, rows_v, *bsems):
    base = _worker_base()
    lanes = lax.iota(jnp.int32, 16)

    def sidx(e):
        return idx_v[pl.ds(e, 16)][0]

    def fire(tblT, e, s):
        i = sidx(e)
        start = pl.multiple_of((i // 128) * 128, 128)
        pltpu.async_copy(tblT.at[:, pl.ds(start, 128)],
                         blk_v.at[s], bsems[s])

    def drain(tblT, s):
        pltpu.make_async_copy(tblT.at[:, pl.ds(0, 128)], blk_v.at[s],
                              bsems[s]).wait()

    def extract(e, s):
        lane = jnp.full((16,), sidx(e) % 128, jnp.int32)
        rows_v[e, pl.ds(0, 16)] = plsc.load_gather(
            blk_v.at[s], [lanes, lane])
        rows_v[e, pl.ds(16, 16)] = plsc.load_gather(
            blk_v.at[s], [lanes + 16, lane])

    pltpu.sync_copy(item_hbm.at[pl.ds(base, BPW)], idx_v.at[pl.ds(0, BPW)])
    for tblT, o in ((giT, o_gi), (tiT, o_ti)):
        for s in range(MF_NBUF):
            fire(tblT, s, s)

        def group(g, _, tblT=tblT):
            for s in range(MF_NBUF):
                e = g * MF_NBUF + s
                drain(tblT, s)
                extract(e, s)

                @pl.when(e + MF_NBUF < BPW)
                def _(e=e, s=s, tblT=tblT):
                    fire(tblT, e + MF_NBUF, s)
            return 0

        lax.fori_loop(0, BPW // MF_NBUF, group, 0)
        pltpu.sync_copy(rows_v, o.at[pl.ds(base, BPW)])


@jax.jit
def _encoder_call(user, item, user_doc2, item_doc2,
                  gamma_user_w, gamma_item_w, theta_user_w, theta_item_w,
                  doc_w):
    mesh = plsc.VectorSubcoreMesh(core_axis_name="c", subcore_axis_name="s",
                                  num_cores=NC, num_subcores=NS)

    doc_out = (
        jax.ShapeDtypeStruct((B, MF_DIM), jnp.float32),
        jax.ShapeDtypeStruct((B, MF_DIM), jnp.float32),
        jax.ShapeDtypeStruct((B, WORD_DIM), jnp.float32),
        jax.ShapeDtypeStruct((B, WORD_DIM), jnp.float32),
    )
    doc_scratch = [
        pltpu.VMEM((BPW,), jnp.int32),
        pltpu.VMEM((BPW, MF_DIM), jnp.float32),
        pltpu.VMEM((BPW, MF_DIM), jnp.float32),
        pltpu.VMEM((2 * BPW, HALF), jnp.int32),
        pltpu.VMEM((DOC_NBUF, DOC_LEN, WORD_DIM), jnp.float32),
        pltpu.VMEM((BPW, WORD_DIM), jnp.float32),
    ] + [pltpu.SemaphoreType.DMA] * (1 + DOC_NBUF)
    doc_run = pl.kernel(_doc_body, out_type=doc_out, mesh=mesh,
                        scratch_types=doc_scratch,
                        compiler_params=pltpu.CompilerParams(
                            use_tc_tiling_on_sc=False))
    gu, tu, ud_embed, id_embed = doc_run(
        user, user_doc2, item_doc2, gamma_user_w, theta_user_w, doc_w)

    item_out = (
        jax.ShapeDtypeStruct((B, MF_DIM), jnp.float32),
        jax.ShapeDtypeStruct((B, MF_DIM), jnp.float32),
    )
    item_scratch = [
        pltpu.VMEM((BPW + 16,), jnp.int32),
        pltpu.VMEM((MF_NBUF, MF_DIM, 128), jnp.float32),
        pltpu.VMEM((BPW, MF_DIM), jnp.float32),
    ] + [pltpu.SemaphoreType.DMA] * MF_NBUF
    item_run = pl.kernel(_item_body, out_type=item_out, mesh=mesh,
                         scratch_types=item_scratch,
                         compiler_params=pltpu.CompilerParams(
                             use_tc_tiling_on_sc=True,
                             disable_bounds_checks=True,
                             needs_layout_passes=False))
    gi, ti = item_run(item, gamma_item_w.T, theta_item_w.T)
    return gu, gi, tu, ti, ud_embed, id_embed


def kernel(user, item, user_doc, item_doc, gamma_user_w, gamma_item_w,
           theta_user_w, theta_item_w, doc_w):
    user_doc2 = user_doc.reshape(2 * B, HALF)
    item_doc2 = item_doc.reshape(2 * B, HALF)
    return _encoder_call(user, item, user_doc2, item_doc2,
                         gamma_user_w, gamma_item_w, theta_user_w,
                         theta_item_w, doc_w)


# speedup vs baseline: 17.4267x; 1.0724x over previous
import functools

import jax
import jax.numpy as jnp
from jax import lax
from jax.experimental import pallas as pl
from jax.experimental.pallas import tpu as pltpu
from jax.experimental.pallas import tpu_sc as plsc

MF_DIM = 32
WORD_DIM = 64
DOC_LEN = 200
HALF = DOC_LEN // 2
B = 4096
NC, NS = 2, 16
NW = NC * NS
BPW = B // NW
SCALE = 1.0 / DOC_LEN
UNROLL = 10
DOC_NBUF = 4
MF_NBUF = 8


def _worker_base():
    wid = lax.axis_index("s") * NC + lax.axis_index("c")
    return wid * BPW


def _accum_doc(rows_ref):
    zero = jnp.zeros((16,), jnp.float32)

    def body(j, acc):
        row0 = j * UNROLL
        acc = list(acc)
        for r in range(UNROLL):
            for c in range(4):
                k = (r % 2) * 4 + c
                acc[k] = acc[k] + rows_ref[row0 + r, pl.ds(c * 16, 16)]
        return tuple(acc)

    acc = lax.fori_loop(0, DOC_LEN // UNROLL, body, (zero,) * 8)
    return tuple(acc[c] + acc[4 + c] for c in range(4))


def _doc_body(user_doc_hbm, item_doc_hbm, doc_w,
              o_ud, o_id,
              doc_idx_v, doc_rows_v, out_doc_v, *dsems):
    base = _worker_base()

    def fire(e, s):
        buf = doc_rows_v.at[s]
        pltpu.async_copy(doc_w.at[doc_idx_v.at[2 * e]],
                         buf.at[pl.ds(0, HALF)], dsems[s])
        pltpu.async_copy(doc_w.at[doc_idx_v.at[2 * e + 1]],
                         buf.at[pl.ds(HALF, HALF)], dsems[s])

    def drain(s):
        pltpu.make_async_copy(doc_w.at[pl.ds(0, DOC_LEN)], doc_rows_v.at[s],
                              dsems[s]).wait()

    def pool(e, s):
        a0, a1, a2, a3 = _accum_doc(doc_rows_v.at[s])
        out_doc_v[e, pl.ds(0, 16)] = a0 * SCALE
        out_doc_v[e, pl.ds(16, 16)] = a1 * SCALE
        out_doc_v[e, pl.ds(32, 16)] = a2 * SCALE
        out_doc_v[e, pl.ds(48, 16)] = a3 * SCALE

    for doc_hbm, o_doc in ((user_doc_hbm, o_ud), (item_doc_hbm, o_id)):
        pltpu.sync_copy(doc_hbm.at[pl.ds(base * 2, 2 * BPW)], doc_idx_v)
        for s in range(DOC_NBUF):
            fire(s, s)

        def group(g, _):
            for s in range(DOC_NBUF):
                e = g * DOC_NBUF + s
                drain(s)
                pool(e, s)

                @pl.when(e + DOC_NBUF < BPW)
                def _(e=e, s=s):
                    fire(e + DOC_NBUF, s)
            return 0

        lax.fori_loop(0, BPW // DOC_NBUF, group, 0)
        pltpu.sync_copy(out_doc_v, o_doc.at[pl.ds(base, BPW)])


def _mf_body(user_hbm, item_hbm, guT, giT, tuT, tiT,
             o_gu, o_gi, o_tu, o_ti,
             uidx_v, iidx_v, blk_v, rows_v, *bsems):
    base = _worker_base()
    lanes = lax.iota(jnp.int32, 16)

    def sidx(idx_v, e):
        return idx_v[pl.ds(e, 16)][0]

    def fire(idx_v, tblT, e, s):
        i = sidx(idx_v, e)
        start = pl.multiple_of((i // 128) * 128, 128)
        pltpu.async_copy(tblT.at[:, pl.ds(start, 128)],
                         blk_v.at[s], bsems[s])

    def drain(tblT, s):
        pltpu.make_async_copy(tblT.at[:, pl.ds(0, 128)], blk_v.at[s],
                              bsems[s]).wait()

    def extract(idx_v, e, s):
        lane = jnp.full((16,), sidx(idx_v, e) % 128, jnp.int32)
        rows_v[e, pl.ds(0, 16)] = plsc.load_gather(
            blk_v.at[s], [lanes, lane])
        rows_v[e, pl.ds(16, 16)] = plsc.load_gather(
            blk_v.at[s], [lanes + 16, lane])

    pltpu.sync_copy(user_hbm.at[pl.ds(base, BPW)], uidx_v.at[pl.ds(0, BPW)])
    pltpu.sync_copy(item_hbm.at[pl.ds(base, BPW)], iidx_v.at[pl.ds(0, BPW)])
    for idx_v, tblT, o in ((uidx_v, guT, o_gu), (iidx_v, giT, o_gi),
                           (uidx_v, tuT, o_tu), (iidx_v, tiT, o_ti)):
        for s in range(MF_NBUF):
            fire(idx_v, tblT, s, s)

        def group(g, _, idx_v=idx_v, tblT=tblT):
            for s in range(MF_NBUF):
                e = g * MF_NBUF + s
                drain(tblT, s)
                extract(idx_v, e, s)

                @pl.when(e + MF_NBUF < BPW)
                def _(e=e, s=s, idx_v=idx_v, tblT=tblT):
                    fire(idx_v, tblT, e + MF_NBUF, s)
            return 0

        lax.fori_loop(0, BPW // MF_NBUF, group, 0)
        pltpu.sync_copy(rows_v, o.at[pl.ds(base, BPW)])


@jax.jit
def _encoder_call(user, item, user_doc2, item_doc2,
                  gamma_user_w, gamma_item_w, theta_user_w, theta_item_w,
                  doc_w):
    mesh = plsc.VectorSubcoreMesh(core_axis_name="c", subcore_axis_name="s",
                                  num_cores=NC, num_subcores=NS)

    doc_out = (
        jax.ShapeDtypeStruct((B, WORD_DIM), jnp.float32),
        jax.ShapeDtypeStruct((B, WORD_DIM), jnp.float32),
    )
    doc_scratch = [
        pltpu.VMEM((2 * BPW, HALF), jnp.int32),
        pltpu.VMEM((DOC_NBUF, DOC_LEN, WORD_DIM), jnp.float32),
        pltpu.VMEM((BPW, WORD_DIM), jnp.float32),
    ] + [pltpu.SemaphoreType.DMA] * DOC_NBUF
    doc_run = pl.kernel(_doc_body, out_type=doc_out, mesh=mesh,
                        scratch_types=doc_scratch,
                        compiler_params=pltpu.CompilerParams(
                            use_tc_tiling_on_sc=False))
    ud_embed, id_embed = doc_run(user_doc2, item_doc2, doc_w)

    mf_out = tuple(
        jax.ShapeDtypeStruct((B, MF_DIM), jnp.float32) for _ in range(4))
    mf_scratch = [
        pltpu.VMEM((BPW + 16,), jnp.int32),
        pltpu.VMEM((BPW + 16,), jnp.int32),
        pltpu.VMEM((MF_NBUF, MF_DIM, 128), jnp.float32),
        pltpu.VMEM((BPW, MF_DIM), jnp.float32),
    ] + [pltpu.SemaphoreType.DMA] * MF_NBUF
    mf_run = pl.kernel(_mf_body, out_type=mf_out, mesh=mesh,
                       scratch_types=mf_scratch,
                       compiler_params=pltpu.CompilerParams(
                           use_tc_tiling_on_sc=True,
                           disable_bounds_checks=True,
                           needs_layout_passes=False))
    gu, gi, tu, ti = mf_run(user, item, gamma_user_w.T, gamma_item_w.T,
                            theta_user_w.T, theta_item_w.T)
    return gu, gi, tu, ti, ud_embed, id_embed


def kernel(user, item, user_doc, item_doc, gamma_user_w, gamma_item_w,
           theta_user_w, theta_item_w, doc_w):
    user_doc2 = user_doc.reshape(2 * B, HALF)
    item_doc2 = item_doc.reshape(2 * B, HALF)
    return _encoder_call(user, item, user_doc2, item_doc2,
                         gamma_user_w, gamma_item_w, theta_user_w,
                         theta_item_w, doc_w)
